# baseline (speedup 1.0000x reference)
import functools

import jax
import jax.numpy as jnp
from jax import lax
from jax.experimental import pallas as pl
from jax.experimental.pallas import tpu as pltpu
from jax.experimental.pallas import tpu_sc as plsc

N = 4096
D = 512
H = 512
TOPK = 6
KPAD = 8
SCALE = 512 ** (-0.5)
NEG = -1e30

BLK_A = 512
BLK_C = 256
BLK_E = 256


def _lrelu(v):
    return jnp.where(v > 0, v, 0.01 * v)


def _proj1_body(x_ref, w_ref, b_ref, x1_ref, sum_ref):
    i = pl.program_id(0)
    x1 = _lrelu(jnp.dot(x_ref[...], w_ref[...],
                        preferred_element_type=jnp.float32) + b_ref[...])
    x1_ref[...] = x1

    @pl.when(i == 0)
    def _():
        sum_ref[...] = jnp.zeros_like(sum_ref)

    sum_ref[...] += jnp.sum(x1, axis=0, keepdims=True)


def _stage1(x2d, W1, b1r):
    grid = N // BLK_A
    return pl.pallas_call(
        _proj1_body,
        grid=(grid,),
        in_specs=[
            pl.BlockSpec((BLK_A, D), lambda i: (i, 0)),
            pl.BlockSpec((D, H), lambda i: (0, 0)),
            pl.BlockSpec((1, H), lambda i: (0, 0)),
        ],
        out_specs=[
            pl.BlockSpec((BLK_A, H), lambda i: (i, 0)),
            pl.BlockSpec((1, H), lambda i: (0, 0)),
        ],
        out_shape=[
            jax.ShapeDtypeStruct((N, H), jnp.float32),
            jax.ShapeDtypeStruct((1, H), jnp.float32),
        ],
    )(x2d, W1, b1r)


def _proj2_body(x1_ref, mean_ref, wh_ref, bh_ref, wt_ref, bt_ref,
                eh_ref, et_ref, etp_ref):
    x2 = (x1_ref[...] + mean_ref[...]) * 0.5
    eh_ref[...] = jnp.dot(x2, wh_ref[...],
                          preferred_element_type=jnp.float32) + bh_ref[...]
    et = jnp.dot(x2, wt_ref[...],
                 preferred_element_type=jnp.float32) + bt_ref[...]
    et_ref[...] = et.astype(jnp.bfloat16)
    lo = lax.bitcast_convert_type(
        et[:, :H // 2].astype(jnp.bfloat16).astype(jnp.float32), jnp.int32)
    hi = lax.bitcast_convert_type(
        et[:, H // 2:].astype(jnp.bfloat16).astype(jnp.float32), jnp.int32)
    etp_ref[...] = ((lo >> 16) & jnp.int32(0xFFFF)) | hi


def _stage2(x1, mean, Wh, bhr, Wt, btr):
    grid = N // BLK_A
    return pl.pallas_call(
        _proj2_body,
        grid=(grid,),
        in_specs=[
            pl.BlockSpec((BLK_A, H), lambda i: (i, 0)),
            pl.BlockSpec((1, H), lambda i: (0, 0)),
            pl.BlockSpec((H, H), lambda i: (0, 0)),
            pl.BlockSpec((1, H), lambda i: (0, 0)),
            pl.BlockSpec((H, H), lambda i: (0, 0)),
            pl.BlockSpec((1, H), lambda i: (0, 0)),
        ],
        out_specs=[
            pl.BlockSpec((BLK_A, H), lambda i: (i, 0)),
            pl.BlockSpec((BLK_A, H), lambda i: (i, 0)),
            pl.BlockSpec((BLK_A, H // 2), lambda i: (i, 0)),
        ],
        out_shape=[
            jax.ShapeDtypeStruct((N, H), jnp.float32),
            jax.ShapeDtypeStruct((N, H), jnp.bfloat16),
            jax.ShapeDtypeStruct((N, H // 2), jnp.int32),
        ],
    )(x1, mean, Wh, bhr, Wt, btr)


def _topk_body(eh_ref, et_ref, tw_ref, ti_ref):
    logits = lax.dot_general(
        (eh_ref[...] * SCALE).astype(jnp.bfloat16), et_ref[...],
        dimension_numbers=(((1,), (1,)), ((), ())),
        preferred_element_type=jnp.float32)
    cols = lax.broadcasted_iota(jnp.int32, (BLK_C, N), 1)
    vals = []
    idxs = []
    for _ in range(TOPK):
        m = jnp.max(logits, axis=1, keepdims=True)
        eq = logits == m
        idx = jnp.min(jnp.where(eq, cols, jnp.int32(2**30)),
                      axis=1, keepdims=True)
        logits = jnp.where(cols == idx, NEG, logits)
        vals.append(m)
        idxs.append(idx)
    vals += [jnp.full((BLK_C, 1), NEG, jnp.float32)] * (KPAD - TOPK)
    idxs += [jnp.zeros((BLK_C, 1), jnp.int32)] * (KPAD - TOPK)
    tw_ref[...] = jnp.concatenate(vals, axis=1)
    ti_ref[...] = jnp.concatenate(idxs, axis=1)


def _stage3(e_h, e_t, half):
    hn = N // 2
    grid = hn // BLK_C
    off = half * grid
    return pl.pallas_call(
        _topk_body,
        grid=(grid,),
        in_specs=[
            pl.BlockSpec((BLK_C, H), lambda i: (i + off, 0)),
            pl.BlockSpec((N, H), lambda i: (0, 0)),
        ],
        out_specs=[
            pl.BlockSpec((BLK_C, KPAD), lambda i: (i, 0)),
            pl.BlockSpec((BLK_C, KPAD), lambda i: (i, 0)),
        ],
        out_shape=[
            jax.ShapeDtypeStruct((hn, KPAD), jnp.float32),
            jax.ShapeDtypeStruct((hn, KPAD), jnp.int32),
        ],
    )(e_h, e_t)


_SC_WORKERS = 32
_SC_CHUNK = 128
_SC_NBUF = 3


def _sc_gather_body(nchunk, per_w, et_hbm, idx_hbm, out_hbm, idx_all,
                    r0, r1, r2, g0, g1, g2, s0, s1, s2):
    wid = lax.axis_index("s") * 2 + lax.axis_index("c")
    base = wid * per_w
    pltpu.sync_copy(idx_hbm.at[pl.ds(base, per_w)], idx_all)
    bufs = (r0, r1, r2)
    gsems = (g0, g1, g2)
    ssems = (s0, s1, s2)
    gh = [None] * nchunk
    sh = [None] * nchunk

    def start_gather(c):
        b = c % _SC_NBUF
        return pltpu.async_copy(
            et_hbm.at[idx_all.at[pl.ds(c * _SC_CHUNK, _SC_CHUNK)]],
            bufs[b], gsems[b])

    def start_scatter(c):
        b = c % _SC_NBUF
        return pltpu.async_copy(
            bufs[b], out_hbm.at[pl.ds(base + c * _SC_CHUNK, _SC_CHUNK)],
            ssems[b])

    for c in range(nchunk):
        if c >= _SC_NBUF:
            sh[c - _SC_NBUF].wait()
        gh[c] = start_gather(c)
        if c >= 1:
            gh[c - 1].wait()
            sh[c - 1] = start_scatter(c - 1)
    last = nchunk - 1
    gh[last].wait()
    sh[last] = start_scatter(last)
    for c in range(max(0, nchunk - _SC_NBUF), nchunk):
        if sh[c] is not None:
            sh[c].wait()


def _sc_gather(table, idx_flat):
    rows = idx_flat.shape[0]
    width = table.shape[1]
    per_w = rows // _SC_WORKERS
    nchunk = per_w // _SC_CHUNK
    mesh = plsc.VectorSubcoreMesh(core_axis_name="c", subcore_axis_name="s")
    return pl.kernel(
        functools.partial(_sc_gather_body, nchunk, per_w),
        out_type=jax.ShapeDtypeStruct((rows, width), table.dtype),
        mesh=mesh,
        scratch_types=[
            pltpu.VMEM((per_w,), jnp.int32),
            pltpu.VMEM((_SC_CHUNK, width), table.dtype),
            pltpu.VMEM((_SC_CHUNK, width), table.dtype),
            pltpu.VMEM((_SC_CHUNK, width), table.dtype),
            pltpu.SemaphoreType.DMA,
            pltpu.SemaphoreType.DMA,
            pltpu.SemaphoreType.DMA,
            pltpu.SemaphoreType.DMA,
            pltpu.SemaphoreType.DMA,
            pltpu.SemaphoreType.DMA,
        ],
    )(table, idx_flat)


def _agg_body(final, eh_ref, nb0_ref, nb1_ref, nb2_ref, nb3_ref, nb4_ref,
              nb5_ref, tw_ref, wl1_ref, bl1_ref, wl2_ref,
              bl2_ref, wa1_ref, ba1_ref, wa2_ref, min_ref, sin_ref, accin_ref,
              lng_ref, lnb_ref, wfc_ref, bfc_ref,
              out0_ref, out1_ref, out2_ref, m_ref, s_ref, acc_ref):
    nb_refs = (nb0_ref, nb1_ref, nb2_ref, nb3_ref, nb4_ref, nb5_ref)
    i = pl.program_id(0)
    nblk = pl.num_programs(0)
    e_h = eh_ref[...]

    tw = tw_ref[...]
    tmax = jnp.max(tw, axis=1, keepdims=True)
    te = jnp.exp(tw - tmax)
    p = te / jnp.sum(te, axis=1, keepdims=True)

    nbs = []
    kws = []
    for k in range(TOPK):
        pk_i = nb_refs[k][...]
        lo_f = lax.bitcast_convert_type(pk_i << 16, jnp.float32)
        hi_f = lax.bitcast_convert_type(pk_i & jnp.int32(-65536), jnp.float32)
        nbk = jnp.concatenate([lo_f, hi_f], axis=1)
        pk = p[:, k:k + 1]
        gate = jnp.tanh((2.0 - pk) * e_h + pk * nbk)
        kws.append(jnp.sum(nbk, axis=1, keepdims=True)
                   * jnp.sum(gate, axis=1, keepdims=True))
        nbs.append(nbk)
    kw = jnp.concatenate(kws, axis=1)
    kmax = jnp.max(kw, axis=1, keepdims=True)
    ke = jnp.exp(kw - kmax)
    kp = ke / jnp.sum(ke, axis=1, keepdims=True)

    e_nh = kp[:, 0:1] * nbs[0]
    for k in range(1, TOPK):
        e_nh = e_nh + kp[:, k:k + 1] * nbs[k]

    sum_emb = _lrelu(jnp.dot(e_h + e_nh, wl1_ref[...],
                             preferred_element_type=jnp.float32) + bl1_ref[...])
    bi_emb = _lrelu(jnp.dot(e_h * e_nh, wl2_ref[...],
                            preferred_element_type=jnp.float32) + bl2_ref[...])
    h = sum_emb + bi_emb

    a1 = _lrelu(jnp.dot(h, wa1_ref[...],
                        preferred_element_type=jnp.float32) + ba1_ref[...])
    gl = jnp.sum(a1 * wa2_ref[...], axis=1, keepdims=True)

    @pl.when(i == 0)
    def _():
        m_ref[...] = min_ref[...]
        s_ref[...] = sin_ref[...]
        acc_ref[...] = accin_ref[...]

    m_old = m_ref[...]
    bm = jnp.max(gl, keepdims=True).reshape(1, 1)
    m_new = jnp.maximum(m_old, bm)
    corr = jnp.exp(m_old - m_new)
    w = jnp.exp(gl - m_new)
    s_ref[...] = s_ref[...] * corr + jnp.sum(w, keepdims=True).reshape(1, 1)
    acc_ref[...] = acc_ref[...] * corr + jnp.sum(w * h, axis=0, keepdims=True)
    m_ref[...] = m_new

    @pl.when(i == nblk - 1)
    def _():
        if final:
            pooled = acc_ref[...] / s_ref[...]
            mu = jnp.mean(pooled, axis=1, keepdims=True)
            var = jnp.mean((pooled - mu) ** 2, axis=1, keepdims=True)
            hn = ((pooled - mu) * lax.rsqrt(var + 1e-5) * lng_ref[...]
                  + lnb_ref[...])
            out0_ref[...] = jnp.dot(hn, wfc_ref[...],
                                    preferred_element_type=jnp.float32) \
                + bfc_ref[...]
        else:
            out0_ref[...] = m_ref[...]
            out1_ref[...] = s_ref[...]
            out2_ref[...] = acc_ref[...]


def _stage5(e_h, nb, tw, state, Wl1, bl1r, Wl2, bl2r, Wa1, ba1r, wa2r,
            lngr, lnbr, Wfc_p, bfc_p, half, final):
    hn = N // 2
    grid = hn // BLK_E
    off = half * grid
    if final:
        out_specs = [pl.BlockSpec((1, 128), lambda i: (0, 0))] * 3
        out_shape = [jax.ShapeDtypeStruct((1, 128), jnp.float32)] * 3
    else:
        out_specs = [
            pl.BlockSpec((1, 1), lambda i: (0, 0)),
            pl.BlockSpec((1, 1), lambda i: (0, 0)),
            pl.BlockSpec((1, H), lambda i: (0, 0)),
        ]
        out_shape = [
            jax.ShapeDtypeStruct((1, 1), jnp.float32),
            jax.ShapeDtypeStruct((1, 1), jnp.float32),
            jax.ShapeDtypeStruct((1, H), jnp.float32),
        ]
    return pl.pallas_call(
        functools.partial(_agg_body, final),
        grid=(grid,),
        in_specs=[
            pl.BlockSpec((BLK_E, H), lambda i: (i + off, 0)),
        ] + [
            pl.BlockSpec((BLK_E, H // 2),
                         functools.partial(lambda k, i: (k * grid + i, 0), k))
            for k in range(TOPK)
        ] + [
            pl.BlockSpec((BLK_E, KPAD), lambda i: (i, 0)),
            pl.BlockSpec((H, H), lambda i: (0, 0)),
            pl.BlockSpec((1, H), lambda i: (0, 0)),
            pl.BlockSpec((H, H), lambda i: (0, 0)),
            pl.BlockSpec((1, H), lambda i: (0, 0)),
            pl.BlockSpec((H, H // 2), lambda i: (0, 0)),
            pl.BlockSpec((1, H // 2), lambda i: (0, 0)),
            pl.BlockSpec((1, H // 2), lambda i: (0, 0)),
            pl.BlockSpec((1, 1), lambda i: (0, 0)),
            pl.BlockSpec((1, 1), lambda i: (0, 0)),
            pl.BlockSpec((1, H), lambda i: (0, 0)),
            pl.BlockSpec((1, H), lambda i: (0, 0)),
            pl.BlockSpec((1, H), lambda i: (0, 0)),
            pl.BlockSpec((H, 128), lambda i: (0, 0)),
            pl.BlockSpec((1, 128), lambda i: (0, 0)),
        ],
        out_specs=out_specs,
        out_shape=out_shape,
        scratch_shapes=[
            pltpu.VMEM((1, 1), jnp.float32),
            pltpu.VMEM((1, 1), jnp.float32),
            pltpu.VMEM((1, H), jnp.float32),
        ],
    )(e_h, nb, nb, nb, nb, nb, nb, tw, Wl1, bl1r, Wl2, bl2r, Wa1, ba1r, wa2r,
      state[0], state[1], state[2], lngr, lnbr, Wfc_p, bfc_p)


@jax.jit
def kernel(x, W1, b1, Wh, bh, Wt, bt, Wl1, bl1, Wl2, bl2,
           Wa1, ba1, Wa2, ba2, ln_g, ln_b, Wfc, bfc):
    x2d = x[0]
    hn = N // 2
    x1, colsum = _stage1(x2d, W1, b1.reshape(1, H))
    mean = colsum * (1.0 / N)
    e_h, e_t, etp = _stage2(x1, mean, Wh, bh.reshape(1, H),
                            Wt, bt.reshape(1, H))
    Wfc_p = jnp.pad(Wfc, ((0, 0), (0, 128 - Wfc.shape[1])))
    bfc_p = jnp.pad(bfc, (0, 128 - bfc.shape[0])).reshape(1, 128)
    common = (Wl1, bl1.reshape(1, H), Wl2,
              bl2.reshape(1, H), Wa1, ba1.reshape(1, H // 2),
              Wa2.reshape(1, H // 2),
              ln_g.reshape(1, H), ln_b.reshape(1, H), Wfc_p, bfc_p)

    tw0, ti0 = _stage3(e_h, e_t, 0)
    nb0 = _sc_gather(etp, ti0[:, :TOPK].T.reshape(-1))
    tw1, ti1 = _stage3(e_h, e_t, 1)
    nb1 = _sc_gather(etp, ti1[:, :TOPK].T.reshape(-1))

    state0 = (jnp.full((1, 1), NEG, jnp.float32),
              jnp.zeros((1, 1), jnp.float32),
              jnp.zeros((1, H), jnp.float32))
    state1 = _stage5(e_h, nb0, tw0, state0, *common, half=0, final=False)
    out_p, _, _ = _stage5(e_h, nb1, tw1, state1, *common, half=1, final=True)
    return out_p[:, :2]


# speedup vs baseline: 1.0897x; 1.0897x over previous
import functools

import jax
import jax.numpy as jnp
from jax import lax
from jax.experimental import pallas as pl
from jax.experimental.pallas import tpu as pltpu
from jax.experimental.pallas import tpu_sc as plsc

N = 4096
D = 512
H = 512
TOPK = 6
KPAD = 8
SCALE = 512 ** (-0.5)
NEG = -1e30

BLK_A = 512
BLK_C = 256
BLK_E = 256


def _lrelu(v):
    return jnp.where(v > 0, v, 0.01 * v)


def _proj1_body(x_ref, w_ref, b_ref, x1_ref, sum_ref):
    i = pl.program_id(0)
    x1 = _lrelu(jnp.dot(x_ref[...], w_ref[...],
                        preferred_element_type=jnp.float32) + b_ref[...])
    x1_ref[...] = x1

    @pl.when(i == 0)
    def _():
        sum_ref[...] = jnp.zeros_like(sum_ref)

    sum_ref[...] += jnp.sum(x1, axis=0, keepdims=True)


def _stage1(x2d, W1, b1r):
    grid = N // BLK_A
    return pl.pallas_call(
        _proj1_body,
        grid=(grid,),
        in_specs=[
            pl.BlockSpec((BLK_A, D), lambda i: (i, 0)),
            pl.BlockSpec((D, H), lambda i: (0, 0)),
            pl.BlockSpec((1, H), lambda i: (0, 0)),
        ],
        out_specs=[
            pl.BlockSpec((BLK_A, H), lambda i: (i, 0)),
            pl.BlockSpec((1, H), lambda i: (0, 0)),
        ],
        out_shape=[
            jax.ShapeDtypeStruct((N, H), jnp.float32),
            jax.ShapeDtypeStruct((1, H), jnp.float32),
        ],
    )(x2d, W1, b1r)


def _proj2_body(x1_ref, mean_ref, wh_ref, bh_ref, wt_ref, bt_ref,
                eh_ref, et_ref, etp_ref):
    x2 = (x1_ref[...] + mean_ref[...]) * 0.5
    eh_ref[...] = jnp.dot(x2, wh_ref[...],
                          preferred_element_type=jnp.float32) + bh_ref[...]
    et = jnp.dot(x2, wt_ref[...],
                 preferred_element_type=jnp.float32) + bt_ref[...]
    et_ref[...] = et.astype(jnp.bfloat16)
    lo = lax.bitcast_convert_type(
        et[:, :H // 2].astype(jnp.bfloat16).astype(jnp.float32), jnp.int32)
    hi = lax.bitcast_convert_type(
        et[:, H // 2:].astype(jnp.bfloat16).astype(jnp.float32), jnp.int32)
    etp_ref[...] = ((lo >> 16) & jnp.int32(0xFFFF)) | hi


def _stage2(x1, mean, Wh, bhr, Wt, btr):
    grid = N // BLK_A
    return pl.pallas_call(
        _proj2_body,
        grid=(grid,),
        in_specs=[
            pl.BlockSpec((BLK_A, H), lambda i: (i, 0)),
            pl.BlockSpec((1, H), lambda i: (0, 0)),
            pl.BlockSpec((H, H), lambda i: (0, 0)),
            pl.BlockSpec((1, H), lambda i: (0, 0)),
            pl.BlockSpec((H, H), lambda i: (0, 0)),
            pl.BlockSpec((1, H), lambda i: (0, 0)),
        ],
        out_specs=[
            pl.BlockSpec((BLK_A, H), lambda i: (i, 0)),
            pl.BlockSpec((BLK_A, H), lambda i: (i, 0)),
            pl.BlockSpec((BLK_A, H // 2), lambda i: (i, 0)),
        ],
        out_shape=[
            jax.ShapeDtypeStruct((N, H), jnp.float32),
            jax.ShapeDtypeStruct((N, H), jnp.bfloat16),
            jax.ShapeDtypeStruct((N, H // 2), jnp.int32),
        ],
    )(x1, mean, Wh, bhr, Wt, btr)


def _topk_body(eh_ref, et_ref, tw_ref, ti_ref):
    logits = lax.dot_general(
        (eh_ref[...] * SCALE).astype(jnp.bfloat16), et_ref[...],
        dimension_numbers=(((1,), (1,)), ((), ())),
        preferred_element_type=jnp.float32)
    cols = lax.broadcasted_iota(jnp.int32, (BLK_C, N), 1)
    vals = []
    idxs = []
    for _ in range(TOPK):
        m = jnp.max(logits, axis=1, keepdims=True)
        eq = logits == m
        idx = jnp.min(jnp.where(eq, cols, jnp.int32(2**30)),
                      axis=1, keepdims=True)
        logits = jnp.where(cols == idx, NEG, logits)
        vals.append(m)
        idxs.append(idx)
    vals += [jnp.full((BLK_C, 1), NEG, jnp.float32)] * (KPAD - TOPK)
    idxs += [jnp.zeros((BLK_C, 1), jnp.int32)] * (KPAD - TOPK)
    tw_ref[...] = jnp.concatenate(vals, axis=1)
    ti_ref[...] = jnp.concatenate(idxs, axis=1)


NPARTS = 4


def _stage3(e_h, e_t, part):
    hn = N // NPARTS
    grid = hn // BLK_C
    off = part * grid
    return pl.pallas_call(
        _topk_body,
        grid=(grid,),
        in_specs=[
            pl.BlockSpec((BLK_C, H), lambda i: (i + off, 0)),
            pl.BlockSpec((N, H), lambda i: (0, 0)),
        ],
        out_specs=[
            pl.BlockSpec((BLK_C, KPAD), lambda i: (i, 0)),
            pl.BlockSpec((BLK_C, KPAD), lambda i: (i, 0)),
        ],
        out_shape=[
            jax.ShapeDtypeStruct((hn, KPAD), jnp.float32),
            jax.ShapeDtypeStruct((hn, KPAD), jnp.int32),
        ],
    )(e_h, e_t)


_SC_WORKERS = 32
_SC_CHUNK = 64
_SC_NBUF = 3


def _sc_gather_body(nchunk, per_w, et_hbm, idx_hbm, out_hbm, idx_all,
                    r0, r1, r2, g0, g1, g2, s0, s1, s2):
    wid = lax.axis_index("s") * 2 + lax.axis_index("c")
    base = wid * per_w
    pltpu.sync_copy(idx_hbm.at[pl.ds(base, per_w)], idx_all)
    bufs = (r0, r1, r2)
    gsems = (g0, g1, g2)
    ssems = (s0, s1, s2)
    gh = [None] * nchunk
    sh = [None] * nchunk

    def start_gather(c):
        b = c % _SC_NBUF
        return pltpu.async_copy(
            et_hbm.at[idx_all.at[pl.ds(c * _SC_CHUNK, _SC_CHUNK)]],
            bufs[b], gsems[b])

    def start_scatter(c):
        b = c % _SC_NBUF
        return pltpu.async_copy(
            bufs[b], out_hbm.at[pl.ds(base + c * _SC_CHUNK, _SC_CHUNK)],
            ssems[b])

    for c in range(nchunk):
        if c >= _SC_NBUF:
            sh[c - _SC_NBUF].wait()
        gh[c] = start_gather(c)
        if c >= 1:
            gh[c - 1].wait()
            sh[c - 1] = start_scatter(c - 1)
    last = nchunk - 1
    gh[last].wait()
    sh[last] = start_scatter(last)
    for c in range(max(0, nchunk - _SC_NBUF), nchunk):
        if sh[c] is not None:
            sh[c].wait()


def _sc_gather(table, idx_flat):
    rows = idx_flat.shape[0]
    width = table.shape[1]
    per_w = rows // _SC_WORKERS
    nchunk = per_w // _SC_CHUNK
    mesh = plsc.VectorSubcoreMesh(core_axis_name="c", subcore_axis_name="s")
    return pl.kernel(
        functools.partial(_sc_gather_body, nchunk, per_w),
        out_type=jax.ShapeDtypeStruct((rows, width), table.dtype),
        mesh=mesh,
        scratch_types=[
            pltpu.VMEM((per_w,), jnp.int32),
            pltpu.VMEM((_SC_CHUNK, width), table.dtype),
            pltpu.VMEM((_SC_CHUNK, width), table.dtype),
            pltpu.VMEM((_SC_CHUNK, width), table.dtype),
            pltpu.SemaphoreType.DMA,
            pltpu.SemaphoreType.DMA,
            pltpu.SemaphoreType.DMA,
            pltpu.SemaphoreType.DMA,
            pltpu.SemaphoreType.DMA,
            pltpu.SemaphoreType.DMA,
        ],
    )(table, idx_flat)


def _agg_body(final, eh_ref, nb0_ref, nb1_ref, nb2_ref, nb3_ref, nb4_ref,
              nb5_ref, tw_ref, wl1_ref, bl1_ref, wl2_ref,
              bl2_ref, wa1_ref, ba1_ref, wa2_ref, min_ref, sin_ref, accin_ref,
              lng_ref, lnb_ref, wfc_ref, bfc_ref,
              out0_ref, out1_ref, out2_ref, m_ref, s_ref, acc_ref):
    nb_refs = (nb0_ref, nb1_ref, nb2_ref, nb3_ref, nb4_ref, nb5_ref)
    i = pl.program_id(0)
    nblk = pl.num_programs(0)
    e_h = eh_ref[...]

    tw = tw_ref[...]
    tmax = jnp.max(tw, axis=1, keepdims=True)
    te = jnp.exp(tw - tmax)
    p = te / jnp.sum(te, axis=1, keepdims=True)

    nbs = []
    kws = []
    for k in range(TOPK):
        pk_i = nb_refs[k][...]
        lo_f = lax.bitcast_convert_type(pk_i << 16, jnp.float32)
        hi_f = lax.bitcast_convert_type(pk_i & jnp.int32(-65536), jnp.float32)
        nbk = jnp.concatenate([lo_f, hi_f], axis=1)
        pk = p[:, k:k + 1]
        gate = jnp.tanh((2.0 - pk) * e_h + pk * nbk)
        kws.append(jnp.sum(nbk, axis=1, keepdims=True)
                   * jnp.sum(gate, axis=1, keepdims=True))
        nbs.append(nbk)
    kw = jnp.concatenate(kws, axis=1)
    kmax = jnp.max(kw, axis=1, keepdims=True)
    ke = jnp.exp(kw - kmax)
    kp = ke / jnp.sum(ke, axis=1, keepdims=True)

    e_nh = kp[:, 0:1] * nbs[0]
    for k in range(1, TOPK):
        e_nh = e_nh + kp[:, k:k + 1] * nbs[k]

    sum_emb = _lrelu(jnp.dot(e_h + e_nh, wl1_ref[...],
                             preferred_element_type=jnp.float32) + bl1_ref[...])
    bi_emb = _lrelu(jnp.dot(e_h * e_nh, wl2_ref[...],
                            preferred_element_type=jnp.float32) + bl2_ref[...])
    h = sum_emb + bi_emb

    a1 = _lrelu(jnp.dot(h, wa1_ref[...],
                        preferred_element_type=jnp.float32) + ba1_ref[...])
    gl = jnp.sum(a1 * wa2_ref[...], axis=1, keepdims=True)

    @pl.when(i == 0)
    def _():
        m_ref[...] = min_ref[...]
        s_ref[...] = sin_ref[...]
        acc_ref[...] = accin_ref[...]

    m_old = m_ref[...]
    bm = jnp.max(gl, keepdims=True).reshape(1, 1)
    m_new = jnp.maximum(m_old, bm)
    corr = jnp.exp(m_old - m_new)
    w = jnp.exp(gl - m_new)
    s_ref[...] = s_ref[...] * corr + jnp.sum(w, keepdims=True).reshape(1, 1)
    acc_ref[...] = acc_ref[...] * corr + jnp.sum(w * h, axis=0, keepdims=True)
    m_ref[...] = m_new

    @pl.when(i == nblk - 1)
    def _():
        if final:
            pooled = acc_ref[...] / s_ref[...]
            mu = jnp.mean(pooled, axis=1, keepdims=True)
            var = jnp.mean((pooled - mu) ** 2, axis=1, keepdims=True)
            hn = ((pooled - mu) * lax.rsqrt(var + 1e-5) * lng_ref[...]
                  + lnb_ref[...])
            out0_ref[...] = jnp.dot(hn, wfc_ref[...],
                                    preferred_element_type=jnp.float32) \
                + bfc_ref[...]
        else:
            out0_ref[...] = m_ref[...]
            out1_ref[...] = s_ref[...]
            out2_ref[...] = acc_ref[...]


def _stage5(e_h, nb, tw, state, Wl1, bl1r, Wl2, bl2r, Wa1, ba1r, wa2r,
            lngr, lnbr, Wfc_p, bfc_p, part, final):
    hn = N // NPARTS
    grid = hn // BLK_E
    off = part * grid
    if final:
        out_specs = [pl.BlockSpec((1, 128), lambda i: (0, 0))] * 3
        out_shape = [jax.ShapeDtypeStruct((1, 128), jnp.float32)] * 3
    else:
        out_specs = [
            pl.BlockSpec((1, 1), lambda i: (0, 0)),
            pl.BlockSpec((1, 1), lambda i: (0, 0)),
            pl.BlockSpec((1, H), lambda i: (0, 0)),
        ]
        out_shape = [
            jax.ShapeDtypeStruct((1, 1), jnp.float32),
            jax.ShapeDtypeStruct((1, 1), jnp.float32),
            jax.ShapeDtypeStruct((1, H), jnp.float32),
        ]
    return pl.pallas_call(
        functools.partial(_agg_body, final),
        grid=(grid,),
        in_specs=[
            pl.BlockSpec((BLK_E, H), lambda i: (i + off, 0)),
        ] + [
            pl.BlockSpec((BLK_E, H // 2),
                         functools.partial(lambda k, i: (k * grid + i, 0), k))
            for k in range(TOPK)
        ] + [
            pl.BlockSpec((BLK_E, KPAD), lambda i: (i, 0)),
            pl.BlockSpec((H, H), lambda i: (0, 0)),
            pl.BlockSpec((1, H), lambda i: (0, 0)),
            pl.BlockSpec((H, H), lambda i: (0, 0)),
            pl.BlockSpec((1, H), lambda i: (0, 0)),
            pl.BlockSpec((H, H // 2), lambda i: (0, 0)),
            pl.BlockSpec((1, H // 2), lambda i: (0, 0)),
            pl.BlockSpec((1, H // 2), lambda i: (0, 0)),
            pl.BlockSpec((1, 1), lambda i: (0, 0)),
            pl.BlockSpec((1, 1), lambda i: (0, 0)),
            pl.BlockSpec((1, H), lambda i: (0, 0)),
            pl.BlockSpec((1, H), lambda i: (0, 0)),
            pl.BlockSpec((1, H), lambda i: (0, 0)),
            pl.BlockSpec((H, 128), lambda i: (0, 0)),
            pl.BlockSpec((1, 128), lambda i: (0, 0)),
        ],
        out_specs=out_specs,
        out_shape=out_shape,
        scratch_shapes=[
            pltpu.VMEM((1, 1), jnp.float32),
            pltpu.VMEM((1, 1), jnp.float32),
            pltpu.VMEM((1, H), jnp.float32),
        ],
    )(e_h, nb, nb, nb, nb, nb, nb, tw, Wl1, bl1r, Wl2, bl2r, Wa1, ba1r, wa2r,
      state[0], state[1], state[2], lngr, lnbr, Wfc_p, bfc_p)


@jax.jit
def kernel(x, W1, b1, Wh, bh, Wt, bt, Wl1, bl1, Wl2, bl2,
           Wa1, ba1, Wa2, ba2, ln_g, ln_b, Wfc, bfc):
    x2d = x[0]
    hn = N // 2
    x1, colsum = _stage1(x2d, W1, b1.reshape(1, H))
    mean = colsum * (1.0 / N)
    e_h, e_t, etp = _stage2(x1, mean, Wh, bh.reshape(1, H),
                            Wt, bt.reshape(1, H))
    Wfc_p = jnp.pad(Wfc, ((0, 0), (0, 128 - Wfc.shape[1])))
    bfc_p = jnp.pad(bfc, (0, 128 - bfc.shape[0])).reshape(1, 128)
    common = (Wl1, bl1.reshape(1, H), Wl2,
              bl2.reshape(1, H), Wa1, ba1.reshape(1, H // 2),
              Wa2.reshape(1, H // 2),
              ln_g.reshape(1, H), ln_b.reshape(1, H), Wfc_p, bfc_p)

    tws, nbs = [], []
    for q in range(NPARTS):
        tw_q, ti_q = _stage3(e_h, e_t, q)
        tws.append(tw_q)
        nbs.append(_sc_gather(etp, ti_q[:, :TOPK].T.reshape(-1)))

    state = (jnp.full((1, 1), NEG, jnp.float32),
             jnp.zeros((1, 1), jnp.float32),
             jnp.zeros((1, H), jnp.float32))
    for q in range(NPARTS - 1):
        state = _stage5(e_h, nbs[q], tws[q], state, *common,
                        part=q, final=False)
    out_p, _, _ = _stage5(e_h, nbs[-1], tws[-1], state, *common,
                          part=NPARTS - 1, final=True)
    return out_p[:, :2]


# speedup vs baseline: 1.1815x; 1.0842x over previous
import functools

import jax
import jax.numpy as jnp
from jax import lax
from jax.experimental import pallas as pl
from jax.experimental.pallas import tpu as pltpu
from jax.experimental.pallas import tpu_sc as plsc

N = 4096
D = 512
H = 512
TOPK = 6
KPAD = 8
SCALE = 512 ** (-0.5)
NEG = -1e30

BLK_A = 512
BLK_C = 256
BLK_E = 256


def _lrelu(v):
    return jnp.where(v > 0, v, 0.01 * v)


def _proj1_body(x_ref, w_ref, b_ref, x1_ref, sum_ref):
    i = pl.program_id(0)
    x1 = _lrelu(jnp.dot(x_ref[...], w_ref[...],
                        preferred_element_type=jnp.float32) + b_ref[...])
    x1_ref[...] = x1

    @pl.when(i == 0)
    def _():
        sum_ref[...] = jnp.zeros_like(sum_ref)

    sum_ref[...] += jnp.sum(x1, axis=0, keepdims=True)


def _stage1(x2d, W1, b1r):
    grid = N // BLK_A
    return pl.pallas_call(
        _proj1_body,
        grid=(grid,),
        in_specs=[
            pl.BlockSpec((BLK_A, D), lambda i: (i, 0)),
            pl.BlockSpec((D, H), lambda i: (0, 0)),
            pl.BlockSpec((1, H), lambda i: (0, 0)),
        ],
        out_specs=[
            pl.BlockSpec((BLK_A, H), lambda i: (i, 0)),
            pl.BlockSpec((1, H), lambda i: (0, 0)),
        ],
        out_shape=[
            jax.ShapeDtypeStruct((N, H), jnp.float32),
            jax.ShapeDtypeStruct((1, H), jnp.float32),
        ],
    )(x2d, W1, b1r)


def _proj2_body(x1_ref, mean_ref, wh_ref, bh_ref, wt_ref, bt_ref,
                eh_ref, et_ref, etp_ref):
    x2 = (x1_ref[...] + mean_ref[...]) * 0.5
    eh_ref[...] = jnp.dot(x2, wh_ref[...],
                          preferred_element_type=jnp.float32) + bh_ref[...]
    et = jnp.dot(x2, wt_ref[...],
                 preferred_element_type=jnp.float32) + bt_ref[...]
    et_ref[...] = et.astype(jnp.bfloat16)
    lo = lax.bitcast_convert_type(
        et[:, :H // 2].astype(jnp.bfloat16).astype(jnp.float32), jnp.int32)
    hi = lax.bitcast_convert_type(
        et[:, H // 2:].astype(jnp.bfloat16).astype(jnp.float32), jnp.int32)
    etp_ref[...] = ((lo >> 16) & jnp.int32(0xFFFF)) | hi


def _stage2(x1, mean, Wh, bhr, Wt, btr):
    grid = N // BLK_A
    return pl.pallas_call(
        _proj2_body,
        grid=(grid,),
        in_specs=[
            pl.BlockSpec((BLK_A, H), lambda i: (i, 0)),
            pl.BlockSpec((1, H), lambda i: (0, 0)),
            pl.BlockSpec((H, H), lambda i: (0, 0)),
            pl.BlockSpec((1, H), lambda i: (0, 0)),
            pl.BlockSpec((H, H), lambda i: (0, 0)),
            pl.BlockSpec((1, H), lambda i: (0, 0)),
        ],
        out_specs=[
            pl.BlockSpec((BLK_A, H), lambda i: (i, 0)),
            pl.BlockSpec((BLK_A, H), lambda i: (i, 0)),
            pl.BlockSpec((BLK_A, H // 2), lambda i: (i, 0)),
        ],
        out_shape=[
            jax.ShapeDtypeStruct((N, H), jnp.float32),
            jax.ShapeDtypeStruct((N, H), jnp.bfloat16),
            jax.ShapeDtypeStruct((N, H // 2), jnp.int32),
        ],
    )(x1, mean, Wh, bhr, Wt, btr)


def _topk_body(eh_ref, et_ref, tw_ref, ti_ref):
    logits = lax.dot_general(
        (eh_ref[...] * SCALE).astype(jnp.bfloat16), et_ref[...],
        dimension_numbers=(((1,), (1,)), ((), ())),
        preferred_element_type=jnp.float32)
    cols = lax.broadcasted_iota(jnp.int32, (BLK_C, N), 1)
    li = lax.bitcast_convert_type(logits, jnp.int32)
    key0 = jnp.where(li >= 0, li, li ^ jnp.int32(0x7FFFFFFF))
    key = (key0 & jnp.int32(~0xFFF)) | (jnp.int32(N - 1) - cols)
    ms = []
    for _ in range(TOPK):
        m = jnp.max(key, axis=1, keepdims=True)
        key = jnp.where(key == m, jnp.int32(-2**31), key)
        ms.append(m)
    mk = jnp.concatenate(ms, axis=1)
    idx = jnp.int32(N - 1) - (mk & jnp.int32(0xFFF))
    vi = (mk & jnp.int32(~0xFFF)) | jnp.int32(0x800)
    vi = jnp.where(vi >= 0, vi, vi ^ jnp.int32(0x7FFFFFFF))
    val = lax.bitcast_convert_type(vi, jnp.float32)
    pad_v = jnp.full((BLK_C, KPAD - TOPK), NEG, jnp.float32)
    pad_i = jnp.zeros((BLK_C, KPAD - TOPK), jnp.int32)
    tw_ref[...] = jnp.concatenate([val, pad_v], axis=1)
    ti_ref[...] = jnp.concatenate([idx, pad_i], axis=1)


NPARTS = 4


def _stage3(e_h, e_t, part):
    hn = N // NPARTS
    grid = hn // BLK_C
    off = part * grid
    return pl.pallas_call(
        _topk_body,
        grid=(grid,),
        in_specs=[
            pl.BlockSpec((BLK_C, H), lambda i: (i + off, 0)),
            pl.BlockSpec((N, H), lambda i: (0, 0)),
        ],
        out_specs=[
            pl.BlockSpec((BLK_C, KPAD), lambda i: (i, 0)),
            pl.BlockSpec((BLK_C, KPAD), lambda i: (i, 0)),
        ],
        out_shape=[
            jax.ShapeDtypeStruct((hn, KPAD), jnp.float32),
            jax.ShapeDtypeStruct((hn, KPAD), jnp.int32),
        ],
    )(e_h, e_t)


_SC_WORKERS = 32
_SC_CHUNK = 64
_SC_NBUF = 3


def _sc_gather_body(nchunk, per_w, et_hbm, idx_hbm, out_hbm, idx_all,
                    r0, r1, r2, g0, g1, g2, s0, s1, s2):
    wid = lax.axis_index("s") * 2 + lax.axis_index("c")
    base = wid * per_w
    pltpu.sync_copy(idx_hbm.at[pl.ds(base, per_w)], idx_all)
    bufs = (r0, r1, r2)
    gsems = (g0, g1, g2)
    ssems = (s0, s1, s2)
    gh = [None] * nchunk
    sh = [None] * nchunk

    def start_gather(c):
        b = c % _SC_NBUF
        return pltpu.async_copy(
            et_hbm.at[idx_all.at[pl.ds(c * _SC_CHUNK, _SC_CHUNK)]],
            bufs[b], gsems[b])

    def start_scatter(c):
        b = c % _SC_NBUF
        return pltpu.async_copy(
            bufs[b], out_hbm.at[pl.ds(base + c * _SC_CHUNK, _SC_CHUNK)],
            ssems[b])

    for c in range(nchunk):
        if c >= _SC_NBUF:
            sh[c - _SC_NBUF].wait()
        gh[c] = start_gather(c)
        if c >= 1:
            gh[c - 1].wait()
            sh[c - 1] = start_scatter(c - 1)
    last = nchunk - 1
    gh[last].wait()
    sh[last] = start_scatter(last)
    for c in range(max(0, nchunk - _SC_NBUF), nchunk):
        if sh[c] is not None:
            sh[c].wait()


def _sc_gather(table, idx_flat):
    rows = idx_flat.shape[0]
    width = table.shape[1]
    per_w = rows // _SC_WORKERS
    nchunk = per_w // _SC_CHUNK
    mesh = plsc.VectorSubcoreMesh(core_axis_name="c", subcore_axis_name="s")
    return pl.kernel(
        functools.partial(_sc_gather_body, nchunk, per_w),
        out_type=jax.ShapeDtypeStruct((rows, width), table.dtype),
        mesh=mesh,
        scratch_types=[
            pltpu.VMEM((per_w,), jnp.int32),
            pltpu.VMEM((_SC_CHUNK, width), table.dtype),
            pltpu.VMEM((_SC_CHUNK, width), table.dtype),
            pltpu.VMEM((_SC_CHUNK, width), table.dtype),
            pltpu.SemaphoreType.DMA,
            pltpu.SemaphoreType.DMA,
            pltpu.SemaphoreType.DMA,
            pltpu.SemaphoreType.DMA,
            pltpu.SemaphoreType.DMA,
            pltpu.SemaphoreType.DMA,
        ],
    )(table, idx_flat)


def _agg_body(final, eh_ref, nb0_ref, nb1_ref, nb2_ref, nb3_ref, nb4_ref,
              nb5_ref, tw_ref, wl1_ref, bl1_ref, wl2_ref,
              bl2_ref, wa1_ref, ba1_ref, wa2_ref, min_ref, sin_ref, accin_ref,
              lng_ref, lnb_ref, wfc_ref, bfc_ref,
              out0_ref, out1_ref, out2_ref, m_ref, s_ref, acc_ref):
    nb_refs = (nb0_ref, nb1_ref, nb2_ref, nb3_ref, nb4_ref, nb5_ref)
    i = pl.program_id(0)
    nblk = pl.num_programs(0)
    e_h = eh_ref[...]

    tw = tw_ref[...]
    tmax = jnp.max(tw, axis=1, keepdims=True)
    te = jnp.exp(tw - tmax)
    p = te / jnp.sum(te, axis=1, keepdims=True)

    nbs = []
    kws = []
    for k in range(TOPK):
        pk_i = nb_refs[k][...]
        lo_f = lax.bitcast_convert_type(pk_i << 16, jnp.float32)
        hi_f = lax.bitcast_convert_type(pk_i & jnp.int32(-65536), jnp.float32)
        nbk = jnp.concatenate([lo_f, hi_f], axis=1)
        pk = p[:, k:k + 1]
        gate = jnp.tanh((2.0 - pk) * e_h + pk * nbk)
        kws.append(jnp.sum(nbk, axis=1, keepdims=True)
                   * jnp.sum(gate, axis=1, keepdims=True))
        nbs.append(nbk)
    kw = jnp.concatenate(kws, axis=1)
    kmax = jnp.max(kw, axis=1, keepdims=True)
    ke = jnp.exp(kw - kmax)
    kp = ke / jnp.sum(ke, axis=1, keepdims=True)

    e_nh = kp[:, 0:1] * nbs[0]
    for k in range(1, TOPK):
        e_nh = e_nh + kp[:, k:k + 1] * nbs[k]

    sum_emb = _lrelu(jnp.dot(e_h + e_nh, wl1_ref[...],
                             preferred_element_type=jnp.float32) + bl1_ref[...])
    bi_emb = _lrelu(jnp.dot(e_h * e_nh, wl2_ref[...],
                            preferred_element_type=jnp.float32) + bl2_ref[...])
    h = sum_emb + bi_emb

    a1 = _lrelu(jnp.dot(h, wa1_ref[...],
                        preferred_element_type=jnp.float32) + ba1_ref[...])
    gl = jnp.sum(a1 * wa2_ref[...], axis=1, keepdims=True)

    @pl.when(i == 0)
    def _():
        m_ref[...] = min_ref[...]
        s_ref[...] = sin_ref[...]
        acc_ref[...] = accin_ref[...]

    m_old = m_ref[...]
    bm = jnp.max(gl, keepdims=True).reshape(1, 1)
    m_new = jnp.maximum(m_old, bm)
    corr = jnp.exp(m_old - m_new)
    w = jnp.exp(gl - m_new)
    s_ref[...] = s_ref[...] * corr + jnp.sum(w, keepdims=True).reshape(1, 1)
    acc_ref[...] = acc_ref[...] * corr + jnp.sum(w * h, axis=0, keepdims=True)
    m_ref[...] = m_new

    @pl.when(i == nblk - 1)
    def _():
        if final:
            pooled = acc_ref[...] / s_ref[...]
            mu = jnp.mean(pooled, axis=1, keepdims=True)
            var = jnp.mean((pooled - mu) ** 2, axis=1, keepdims=True)
            hn = ((pooled - mu) * lax.rsqrt(var + 1e-5) * lng_ref[...]
                  + lnb_ref[...])
            out0_ref[...] = jnp.dot(hn, wfc_ref[...],
                                    preferred_element_type=jnp.float32) \
                + bfc_ref[...]
        else:
            out0_ref[...] = m_ref[...]
            out1_ref[...] = s_ref[...]
            out2_ref[...] = acc_ref[...]


def _stage5(e_h, nb, tw, state, Wl1, bl1r, Wl2, bl2r, Wa1, ba1r, wa2r,
            lngr, lnbr, Wfc_p, bfc_p, part, final):
    hn = N // NPARTS
    grid = hn // BLK_E
    off = part * grid
    if final:
        out_specs = [pl.BlockSpec((1, 128), lambda i: (0, 0))] * 3
        out_shape = [jax.ShapeDtypeStruct((1, 128), jnp.float32)] * 3
    else:
        out_specs = [
            pl.BlockSpec((1, 1), lambda i: (0, 0)),
            pl.BlockSpec((1, 1), lambda i: (0, 0)),
            pl.BlockSpec((1, H), lambda i: (0, 0)),
        ]
        out_shape = [
            jax.ShapeDtypeStruct((1, 1), jnp.float32),
            jax.ShapeDtypeStruct((1, 1), jnp.float32),
            jax.ShapeDtypeStruct((1, H), jnp.float32),
        ]
    return pl.pallas_call(
        functools.partial(_agg_body, final),
        grid=(grid,),
        in_specs=[
            pl.BlockSpec((BLK_E, H), lambda i: (i + off, 0)),
        ] + [
            pl.BlockSpec((BLK_E, H // 2),
                         functools.partial(lambda k, i: (k * grid + i, 0), k))
            for k in range(TOPK)
        ] + [
            pl.BlockSpec((BLK_E, KPAD), lambda i: (i, 0)),
            pl.BlockSpec((H, H), lambda i: (0, 0)),
            pl.BlockSpec((1, H), lambda i: (0, 0)),
            pl.BlockSpec((H, H), lambda i: (0, 0)),
            pl.BlockSpec((1, H), lambda i: (0, 0)),
            pl.BlockSpec((H, H // 2), lambda i: (0, 0)),
            pl.BlockSpec((1, H // 2), lambda i: (0, 0)),
            pl.BlockSpec((1, H // 2), lambda i: (0, 0)),
            pl.BlockSpec((1, 1), lambda i: (0, 0)),
            pl.BlockSpec((1, 1), lambda i: (0, 0)),
            pl.BlockSpec((1, H), lambda i: (0, 0)),
            pl.BlockSpec((1, H), lambda i: (0, 0)),
            pl.BlockSpec((1, H), lambda i: (0, 0)),
            pl.BlockSpec((H, 128), lambda i: (0, 0)),
            pl.BlockSpec((1, 128), lambda i: (0, 0)),
        ],
        out_specs=out_specs,
        out_shape=out_shape,
        scratch_shapes=[
            pltpu.VMEM((1, 1), jnp.float32),
            pltpu.VMEM((1, 1), jnp.float32),
            pltpu.VMEM((1, H), jnp.float32),
        ],
    )(e_h, nb, nb, nb, nb, nb, nb, tw, Wl1, bl1r, Wl2, bl2r, Wa1, ba1r, wa2r,
      state[0], state[1], state[2], lngr, lnbr, Wfc_p, bfc_p)


@jax.jit
def kernel(x, W1, b1, Wh, bh, Wt, bt, Wl1, bl1, Wl2, bl2,
           Wa1, ba1, Wa2, ba2, ln_g, ln_b, Wfc, bfc):
    x2d = x[0]
    hn = N // 2
    x1, colsum = _stage1(x2d, W1, b1.reshape(1, H))
    mean = colsum * (1.0 / N)
    e_h, e_t, etp = _stage2(x1, mean, Wh, bh.reshape(1, H),
                            Wt, bt.reshape(1, H))
    Wfc_p = jnp.pad(Wfc, ((0, 0), (0, 128 - Wfc.shape[1])))
    bfc_p = jnp.pad(bfc, (0, 128 - bfc.shape[0])).reshape(1, 128)
    common = (Wl1, bl1.reshape(1, H), Wl2,
              bl2.reshape(1, H), Wa1, ba1.reshape(1, H // 2),
              Wa2.reshape(1, H // 2),
              ln_g.reshape(1, H), ln_b.reshape(1, H), Wfc_p, bfc_p)

    tws, nbs = [], []
    for q in range(NPARTS):
        tw_q, ti_q = _stage3(e_h, e_t, q)
        tws.append(tw_q)
        nbs.append(_sc_gather(etp, ti_q[:, :TOPK].T.reshape(-1)))

    state = (jnp.full((1, 1), NEG, jnp.float32),
             jnp.zeros((1, 1), jnp.float32),
             jnp.zeros((1, H), jnp.float32))
    for q in range(NPARTS - 1):
        state = _stage5(e_h, nbs[q], tws[q], state, *common,
                        part=q, final=False)
    out_p, _, _ = _stage5(e_h, nbs[-1], tws[-1], state, *common,
                          part=NPARTS - 1, final=True)
    return out_p[:, :2]
